# baseline (speedup 1.0000x reference)
import functools

import jax
import jax.numpy as jnp
from jax import lax
from jax.experimental import pallas as pl
from jax.experimental.pallas import tpu as pltpu
from jax.experimental.pallas import tpu_sc as plsc

N = 10000
NPAD = 10112
E = 320000
EPAD = 327680
NC = 2
NS = 16
CH = 80
DEG_NBUF = 4
SCH = 80
NPT = NPAD // NS
ROW_CHUNKS = (128, 128, 128, 128, 120)
ROW_CH = 128

_MESH = dict(core_axis_name="c", subcore_axis_name="s")


@functools.partial(
    pl.kernel,
    mesh=plsc.VectorSubcoreMesh(**_MESH),
    out_type=jax.ShapeDtypeStruct((NC, NPAD, 128), jnp.float32),
    scratch_types=[
        pltpu.VMEM((DEG_NBUF, CH), jnp.int32),
        pltpu.VMEM((CH, 128), jnp.float32),
        pltpu.VMEM((ROW_CH, 128), jnp.float32),
        pltpu.VMEM_SHARED((NPAD, 128), jnp.float32),
    ] + [pltpu.SemaphoreType.DMA] * DEG_NBUF,
)
def _deg_kernel(dst_hbm, deg_hbm, idx_d, ones_v, stage, sh_deg, *sems):
    c = lax.axis_index("c")
    s = lax.axis_index("s")
    ept = EPAD // (NC * NS)
    nb = ept // CH // DEG_NBUF

    one_row = jnp.where(
        lax.broadcasted_iota(jnp.int32, (16,), 0) == 0, 1.0, 0.0
    ).astype(jnp.float32)
    zero = jnp.zeros((16,), jnp.float32)

    def ofill(i, _):
        ones_v[i // 8, pl.ds((i % 8) * 16, 16)] = jnp.where(
            i % 8 == 0, one_row, zero)
        return 0

    lax.fori_loop(0, CH * 8, ofill, 0)

    def zfill(i, _):
        stage[i // 8, pl.ds((i % 8) * 16, 16)] = zero
        return 0

    lax.fori_loop(0, ROW_CH * 8, zfill, 0)
    row0 = s * NPT
    for n in ROW_CHUNKS:
        pltpu.sync_copy(stage.at[pl.ds(0, n)], sh_deg.at[pl.ds(row0, n)])
        row0 += n
    plsc.subcore_barrier()

    wid = s * NC + c

    def body(k2, _):
        for b in range(DEG_NBUF):
            base = wid * ept + (k2 * DEG_NBUF + b) * CH
            pltpu.sync_copy(dst_hbm.at[pl.ds(base, CH)], idx_d.at[b])
            pltpu.async_copy(ones_v, sh_deg.at[idx_d.at[b]], sems[b],
                             add=True)
        for b in range(DEG_NBUF):
            pltpu.make_async_copy(ones_v, sh_deg.at[idx_d.at[b]],
                                  sems[b]).wait()
        return 0

    lax.fori_loop(0, nb, body, 0)
    plsc.subcore_barrier()

    row0 = s * NPT
    for n in ROW_CHUNKS:
        pltpu.sync_copy(sh_deg.at[pl.ds(row0, n)], stage.at[pl.ds(0, n)])
        pltpu.sync_copy(stage.at[pl.ds(0, n)], deg_hbm.at[c].at[pl.ds(row0, n)])
        row0 += n


def _make_spmm(channel_split):
    D = 128
    ept = EPAD // NS if channel_split else EPAD // (NC * NS)
    nchunk = ept // SCH

    @functools.partial(
        pl.kernel,
        mesh=plsc.VectorSubcoreMesh(**_MESH),
        out_type=jax.ShapeDtypeStruct((NC, NPAD, D), jnp.float32),
        scratch_types=[
            pltpu.VMEM((SCH,), jnp.int32),
            pltpu.VMEM((SCH,), jnp.int32),
            pltpu.VMEM((SCH, D), jnp.float32),
            pltpu.VMEM((ROW_CH, D), jnp.float32),
            pltpu.VMEM_SHARED((NPAD, D), jnp.float32),
        ],
    )
    def spmm(u_hbm, src_hbm, dst_hbm, agg_hbm, idx_s, idx_d, rows, stage,
             sh_agg):
        c = lax.axis_index("c")
        s = lax.axis_index("s")

        zero = jnp.zeros((16,), jnp.float32)

        def zfill(i, _):
            stage[i // 8, pl.ds((i % 8) * 16, 16)] = zero
            return 0

        lax.fori_loop(0, ROW_CH * 8, zfill, 0)
        row0 = s * NPT
        for n in ROW_CHUNKS:
            pltpu.sync_copy(stage.at[pl.ds(0, n)], sh_agg.at[pl.ds(row0, n)])
            row0 += n
        plsc.subcore_barrier()

        table = u_hbm.at[c] if channel_split else u_hbm
        wid = s if channel_split else s * NC + c

        def body(k, _):
            base = wid * ept + k * SCH
            pltpu.sync_copy(src_hbm.at[pl.ds(base, SCH)], idx_s)
            pltpu.sync_copy(dst_hbm.at[pl.ds(base, SCH)], idx_d)
            pltpu.sync_copy(table.at[idx_s], rows)
            pltpu.sync_copy(rows, sh_agg.at[idx_d], add=True)
            return 0

        lax.fori_loop(0, nchunk, body, 0)
        plsc.subcore_barrier()

        row0 = s * NPT
        for n in ROW_CHUNKS:
            pltpu.sync_copy(sh_agg.at[pl.ds(row0, n)], stage.at[pl.ds(0, n)])
            pltpu.sync_copy(stage.at[pl.ds(0, n)],
                            agg_hbm.at[c].at[pl.ds(row0, n)])
            row0 += n

    return spmm


_spmm_l1 = _make_spmm(True)
_spmm_l2 = _make_spmm(False)


BLK = 1264


def _tc1(x, W1, degp):
    def body(x_ref, w_ref, deg_ref, u_ref, dinv_ref):
        deg = deg_ref[0, :, 0:1] + deg_ref[1, :, 0:1] + 1.0
        dinv = lax.rsqrt(deg)
        h = jnp.dot(x_ref[...], w_ref[...], preferred_element_type=jnp.float32)
        u = h * dinv
        u_ref[0] = u[:, :128]
        u_ref[1] = u[:, 128:]
        dinv_ref[...] = dinv

    return pl.pallas_call(
        body,
        grid=(NPAD // BLK,),
        in_specs=[
            pl.BlockSpec((BLK, 128), lambda i: (i, 0)),
            pl.BlockSpec((128, 256), lambda i: (0, 0)),
            pl.BlockSpec((2, BLK, 128), lambda i: (0, i, 0)),
        ],
        out_specs=[
            pl.BlockSpec((2, BLK, 128), lambda i: (0, i, 0)),
            pl.BlockSpec((BLK, 1), lambda i: (i, 0)),
        ],
        out_shape=[
            jax.ShapeDtypeStruct((2, NPAD, 128), jnp.float32),
            jax.ShapeDtypeStruct((NPAD, 1), jnp.float32),
        ],
    )(x, W1, degp)


def _tc2(agg1, u1, dinv, b1, W2):
    def body(agg_ref, u_ref, dinv_ref, b_ref, w_ref, u2_ref):
        dinv = dinv_ref[...]
        m = jnp.concatenate(
            [agg_ref[0] + u_ref[0], agg_ref[1] + u_ref[1]], axis=1)
        z = jnp.maximum(dinv * m + b_ref[...], 0.0)
        h2 = jnp.dot(z, w_ref[...], preferred_element_type=jnp.float32)
        u2_ref[...] = dinv * h2

    return pl.pallas_call(
        body,
        grid=(NPAD // BLK,),
        in_specs=[
            pl.BlockSpec((2, BLK, 128), lambda i: (0, i, 0)),
            pl.BlockSpec((2, BLK, 128), lambda i: (0, i, 0)),
            pl.BlockSpec((BLK, 1), lambda i: (i, 0)),
            pl.BlockSpec((1, 256), lambda i: (0, 0)),
            pl.BlockSpec((256, 128), lambda i: (0, 0)),
        ],
        out_specs=pl.BlockSpec((BLK, 128), lambda i: (i, 0)),
        out_shape=jax.ShapeDtypeStruct((NPAD, 128), jnp.float32),
    )(agg1, u1, dinv, b1, W2)


def _tc3(agg2, u2, dinv, b2):
    def body(agg_ref, u_ref, dinv_ref, b_ref, out_ref):
        m = agg_ref[0] + agg_ref[1] + u_ref[...]
        out_ref[...] = dinv_ref[...] * m + b_ref[...]

    return pl.pallas_call(
        body,
        grid=(NPAD // BLK,),
        in_specs=[
            pl.BlockSpec((2, BLK, 128), lambda i: (0, i, 0)),
            pl.BlockSpec((BLK, 128), lambda i: (i, 0)),
            pl.BlockSpec((BLK, 1), lambda i: (i, 0)),
            pl.BlockSpec((1, 128), lambda i: (0, 0)),
        ],
        out_specs=pl.BlockSpec((BLK, 128), lambda i: (i, 0)),
        out_shape=jax.ShapeDtypeStruct((NPAD, 128), jnp.float32),
    )(agg2, u2, dinv, b2)


def kernel(x, edge_index, W1, b1, W2, b2):
    src = jnp.concatenate([edge_index[0].astype(jnp.int32),
                           jnp.zeros((EPAD - E,), jnp.int32)])
    dst = jnp.concatenate([edge_index[1].astype(jnp.int32),
                           jnp.full((EPAD - E,), N, jnp.int32)])
    xp = jnp.concatenate(
        [x, jnp.zeros((NPAD - N, x.shape[1]), x.dtype)], axis=0)
    degp = _deg_kernel(dst)
    u1, dinv = _tc1(xp, W1, degp)
    agg1 = _spmm_l1(u1, src, dst)
    u2 = _tc2(agg1, u1, dinv, b1.reshape(1, -1), W2)
    agg2 = _spmm_l2(u2, src, dst)
    return _tc3(agg2, u2, dinv, b2.reshape(1, -1))[:N]


# speedup vs baseline: 1.0011x; 1.0011x over previous
import functools

import jax
import jax.numpy as jnp
from jax import lax
from jax.experimental import pallas as pl
from jax.experimental.pallas import tpu as pltpu
from jax.experimental.pallas import tpu_sc as plsc

N = 10000
NPAD = 10112
E = 320000
EPAD = 327680
NC = 2
NS = 16
CH = 80
DEG_NBUF = 4
SCH = 80
NPT = NPAD // NS
ROW_CHUNKS = (128, 128, 128, 128, 120)
ROW_CH = 128

_MESH = dict(core_axis_name="c", subcore_axis_name="s")


@functools.partial(
    pl.kernel,
    mesh=plsc.VectorSubcoreMesh(**_MESH),
    out_type=jax.ShapeDtypeStruct((NC, NPAD, 128), jnp.float32),
    scratch_types=[
        pltpu.VMEM((DEG_NBUF, CH), jnp.int32),
        pltpu.VMEM((CH, 128), jnp.float32),
        pltpu.VMEM((ROW_CH, 128), jnp.float32),
        pltpu.VMEM_SHARED((NPAD, 128), jnp.float32),
    ] + [pltpu.SemaphoreType.DMA] * DEG_NBUF,
)
def _deg_kernel(dst_hbm, deg_hbm, idx_d, ones_v, stage, sh_deg, *sems):
    c = lax.axis_index("c")
    s = lax.axis_index("s")
    ept = EPAD // (NC * NS)
    nb = ept // CH // DEG_NBUF

    one_row = jnp.where(
        lax.broadcasted_iota(jnp.int32, (16,), 0) == 0, 1.0, 0.0
    ).astype(jnp.float32)
    zero = jnp.zeros((16,), jnp.float32)

    def ofill(i, _):
        ones_v[i // 8, pl.ds((i % 8) * 16, 16)] = jnp.where(
            i % 8 == 0, one_row, zero)
        return 0

    lax.fori_loop(0, CH * 8, ofill, 0)

    def zfill(i, _):
        stage[i // 8, pl.ds((i % 8) * 16, 16)] = zero
        return 0

    lax.fori_loop(0, ROW_CH * 8, zfill, 0)
    row0 = s * NPT
    for n in ROW_CHUNKS:
        pltpu.sync_copy(stage.at[pl.ds(0, n)], sh_deg.at[pl.ds(row0, n)])
        row0 += n
    plsc.subcore_barrier()

    wid = s * NC + c

    def body(k2, _):
        for b in range(DEG_NBUF):
            base = wid * ept + (k2 * DEG_NBUF + b) * CH
            pltpu.sync_copy(dst_hbm.at[pl.ds(base, CH)], idx_d.at[b])
            pltpu.async_copy(ones_v, sh_deg.at[idx_d.at[b]], sems[b],
                             add=True)
        for b in range(DEG_NBUF):
            pltpu.make_async_copy(ones_v, sh_deg.at[idx_d.at[b]],
                                  sems[b]).wait()
        return 0

    lax.fori_loop(0, nb, body, 0)
    plsc.subcore_barrier()

    row0 = s * NPT
    for n in ROW_CHUNKS:
        pltpu.sync_copy(sh_deg.at[pl.ds(row0, n)], stage.at[pl.ds(0, n)])
        pltpu.sync_copy(stage.at[pl.ds(0, n)], deg_hbm.at[c].at[pl.ds(row0, n)])
        row0 += n


def _make_spmm(channel_split):
    D = 128
    ept = EPAD // NS if channel_split else EPAD // (NC * NS)
    nchunk = ept // SCH

    @functools.partial(
        pl.kernel,
        mesh=plsc.VectorSubcoreMesh(**_MESH),
        out_type=jax.ShapeDtypeStruct((NC, NPAD, D), jnp.float32),
        scratch_types=[
            pltpu.VMEM((SCH,), jnp.int32),
            pltpu.VMEM((SCH,), jnp.int32),
            pltpu.VMEM((SCH, D), jnp.float32),
            pltpu.VMEM((ROW_CH, D), jnp.float32),
            pltpu.VMEM_SHARED((NPAD, D), jnp.float32),
        ],
    )
    def spmm(u_hbm, src_hbm, dst_hbm, agg_hbm, idx_s, idx_d, rows, stage,
             sh_agg):
        c = lax.axis_index("c")
        s = lax.axis_index("s")

        zero = jnp.zeros((16,), jnp.float32)

        def zfill(i, _):
            stage[i // 8, pl.ds((i % 8) * 16, 16)] = zero
            return 0

        lax.fori_loop(0, ROW_CH * 8, zfill, 0)
        row0 = s * NPT
        for n in ROW_CHUNKS:
            pltpu.sync_copy(stage.at[pl.ds(0, n)], sh_agg.at[pl.ds(row0, n)])
            row0 += n
        plsc.subcore_barrier()

        table = u_hbm.at[c] if channel_split else u_hbm
        wid = s if channel_split else s * NC + c

        def body(k, _):
            base = wid * ept + k * SCH
            pltpu.sync_copy(src_hbm.at[pl.ds(base, SCH)], idx_s)
            pltpu.sync_copy(dst_hbm.at[pl.ds(base, SCH)], idx_d)
            pltpu.sync_copy(table.at[idx_s], rows)
            pltpu.sync_copy(rows, sh_agg.at[idx_d], add=True)
            return 0

        lax.fori_loop(0, nchunk, body, 0)
        plsc.subcore_barrier()

        row0 = s * NPT
        for n in ROW_CHUNKS:
            pltpu.sync_copy(sh_agg.at[pl.ds(row0, n)], stage.at[pl.ds(0, n)])
            pltpu.sync_copy(stage.at[pl.ds(0, n)],
                            agg_hbm.at[c].at[pl.ds(row0, n)])
            row0 += n

    return spmm


_spmm_l1 = _make_spmm(True)
_spmm_l2 = _make_spmm(False)


BLK = 1264


def _tc1(x, W1, degp):
    def body(x_ref, w_ref, deg_ref, u_ref, dinv_ref):
        deg = deg_ref[0, :, 0:1] + deg_ref[1, :, 0:1] + 1.0
        dinv = lax.rsqrt(deg)
        h = jnp.dot(x_ref[...], w_ref[...], preferred_element_type=jnp.float32)
        u = h * dinv
        u_ref[0] = u[:, :128]
        u_ref[1] = u[:, 128:]
        dinv_ref[...] = dinv

    return pl.pallas_call(
        body,
        grid=(NPAD // BLK,),
        in_specs=[
            pl.BlockSpec((BLK, 128), lambda i: (i, 0)),
            pl.BlockSpec((128, 256), lambda i: (0, 0)),
            pl.BlockSpec((2, BLK, 128), lambda i: (0, i, 0)),
        ],
        out_specs=[
            pl.BlockSpec((2, BLK, 128), lambda i: (0, i, 0)),
            pl.BlockSpec((BLK, 1), lambda i: (i, 0)),
        ],
        out_shape=[
            jax.ShapeDtypeStruct((2, NPAD, 128), jnp.float32),
            jax.ShapeDtypeStruct((NPAD, 1), jnp.float32),
        ],
    )(x, W1, degp)


def _tc2(agg1, u1, dinv, b1, W2):
    def body(agg_ref, u_ref, dinv_ref, b_ref, w_ref, u2_ref):
        dinv = dinv_ref[...]
        m = jnp.concatenate(
            [agg_ref[0] + u_ref[0], agg_ref[1] + u_ref[1]], axis=1)
        z = jnp.maximum(dinv * m + b_ref[...], 0.0)
        h2 = jnp.dot(z, w_ref[...], preferred_element_type=jnp.float32)
        u2_ref[...] = dinv * h2

    return pl.pallas_call(
        body,
        grid=(NPAD // BLK,),
        in_specs=[
            pl.BlockSpec((2, BLK, 128), lambda i: (0, i, 0)),
            pl.BlockSpec((2, BLK, 128), lambda i: (0, i, 0)),
            pl.BlockSpec((BLK, 1), lambda i: (i, 0)),
            pl.BlockSpec((1, 256), lambda i: (0, 0)),
            pl.BlockSpec((256, 128), lambda i: (0, 0)),
        ],
        out_specs=pl.BlockSpec((BLK, 128), lambda i: (i, 0)),
        out_shape=jax.ShapeDtypeStruct((NPAD, 128), jnp.float32),
    )(agg1, u1, dinv, b1, W2)


def _tc3(agg2, u2, dinv, b2):
    def body(agg_ref, u_ref, dinv_ref, b_ref, out_ref):
        m = agg_ref[0] + agg_ref[1] + u_ref[...]
        out_ref[...] = dinv_ref[...] * m + b_ref[...]

    return pl.pallas_call(
        body,
        grid=(NPAD // BLK,),
        in_specs=[
            pl.BlockSpec((2, BLK, 128), lambda i: (0, i, 0)),
            pl.BlockSpec((BLK, 128), lambda i: (i, 0)),
            pl.BlockSpec((BLK, 1), lambda i: (i, 0)),
            pl.BlockSpec((1, 128), lambda i: (0, 0)),
        ],
        out_specs=pl.BlockSpec((BLK, 128), lambda i: (i, 0)),
        out_shape=jax.ShapeDtypeStruct((NPAD, 128), jnp.float32),
    )(agg2, u2, dinv, b2)


def kernel(x, edge_index, W1, b1, W2, b2):
    src = jnp.concatenate([edge_index[0].astype(jnp.int32),
                           jnp.zeros((EPAD - E,), jnp.int32)])
    dst = jnp.concatenate([edge_index[1].astype(jnp.int32),
                           N + jnp.arange(EPAD - E, dtype=jnp.int32)
                           % (NPAD - N)])
    xp = jnp.concatenate(
        [x, jnp.zeros((NPAD - N, x.shape[1]), x.dtype)], axis=0)
    degp = _deg_kernel(dst)
    u1, dinv = _tc1(xp, W1, degp)
    agg1 = _spmm_l1(u1, src, dst)
    u2 = _tc2(agg1, u1, dinv, b1.reshape(1, -1), W2)
    agg2 = _spmm_l2(u2, src, dst)
    return _tc3(agg2, u2, dinv, b2.reshape(1, -1))[:N]


# speedup vs baseline: 1.5960x; 1.5943x over previous
import functools

import jax
import jax.numpy as jnp
from jax import lax
from jax.experimental import pallas as pl
from jax.experimental.pallas import tpu as pltpu
from jax.experimental.pallas import tpu_sc as plsc

N = 10000
NPAD = 10112
E = 320000
NC = 2
NS = 16
CH = 80
DEG_NBUF = 5
SCH = 80
NPT = NPAD // NS
ROW_CHUNKS = (128, 128, 128, 128, 120)
ROW_CH = 128

_MESH = dict(core_axis_name="c", subcore_axis_name="s")


@functools.partial(
    pl.kernel,
    mesh=plsc.VectorSubcoreMesh(**_MESH),
    out_type=jax.ShapeDtypeStruct((NC, NPAD, 128), jnp.float32),
    scratch_types=[
        pltpu.VMEM((DEG_NBUF, CH), jnp.int32),
        pltpu.VMEM((CH, 128), jnp.float32),
        pltpu.VMEM((ROW_CH, 128), jnp.float32),
        pltpu.VMEM_SHARED((NPAD, 128), jnp.float32),
    ] + [pltpu.SemaphoreType.DMA] * DEG_NBUF,
)
def _deg_kernel(dst_hbm, deg_hbm, idx_d, ones_v, stage, sh_deg, *sems):
    c = lax.axis_index("c")
    s = lax.axis_index("s")
    ept = E // (NC * NS)
    nb = ept // CH // DEG_NBUF

    one_row = jnp.where(
        lax.broadcasted_iota(jnp.int32, (16,), 0) == 0, 1.0, 0.0
    ).astype(jnp.float32)
    zero = jnp.zeros((16,), jnp.float32)

    def ofill(i, _):
        ones_v[i // 8, pl.ds((i % 8) * 16, 16)] = jnp.where(
            i % 8 == 0, one_row, zero)
        return 0

    lax.fori_loop(0, CH * 8, ofill, 0)

    def zfill(i, _):
        stage[i // 8, pl.ds((i % 8) * 16, 16)] = zero
        return 0

    lax.fori_loop(0, ROW_CH * 8, zfill, 0)
    row0 = s * NPT
    for n in ROW_CHUNKS:
        pltpu.sync_copy(stage.at[pl.ds(0, n)], sh_deg.at[pl.ds(row0, n)])
        row0 += n
    plsc.subcore_barrier()

    wid = s * NC + c

    def body(k2, _):
        for b in range(DEG_NBUF):
            base = wid * ept + (k2 * DEG_NBUF + b) * CH
            pltpu.sync_copy(dst_hbm.at[pl.ds(base, CH)], idx_d.at[b])
            pltpu.async_copy(ones_v, sh_deg.at[idx_d.at[b]], sems[b],
                             add=True)
        for b in range(DEG_NBUF):
            pltpu.make_async_copy(ones_v, sh_deg.at[idx_d.at[b]],
                                  sems[b]).wait()
        return 0

    lax.fori_loop(0, nb, body, 0)
    plsc.subcore_barrier()

    row0 = s * NPT
    for n in ROW_CHUNKS:
        pltpu.sync_copy(sh_deg.at[pl.ds(row0, n)], stage.at[pl.ds(0, n)])
        pltpu.sync_copy(stage.at[pl.ds(0, n)], deg_hbm.at[c].at[pl.ds(row0, n)])
        row0 += n


def _make_spmm(channel_split):
    D = 128
    ept = E // NS if channel_split else E // (NC * NS)
    nchunk = ept // SCH

    @functools.partial(
        pl.kernel,
        mesh=plsc.VectorSubcoreMesh(**_MESH),
        out_type=jax.ShapeDtypeStruct((NC, NPAD, D), jnp.float32),
        scratch_types=[
            pltpu.VMEM((SCH,), jnp.int32),
            pltpu.VMEM((SCH,), jnp.int32),
            pltpu.VMEM((SCH, D), jnp.float32),
            pltpu.VMEM((ROW_CH, D), jnp.float32),
            pltpu.VMEM_SHARED((NPAD, D), jnp.float32),
        ],
    )
    def spmm(u_hbm, src_hbm, dst_hbm, agg_hbm, idx_s, idx_d, rows, stage,
             sh_agg):
        c = lax.axis_index("c")
        s = lax.axis_index("s")

        zero = jnp.zeros((16,), jnp.float32)

        def zfill(i, _):
            stage[i // 8, pl.ds((i % 8) * 16, 16)] = zero
            return 0

        lax.fori_loop(0, ROW_CH * 8, zfill, 0)
        row0 = s * NPT
        for n in ROW_CHUNKS:
            pltpu.sync_copy(stage.at[pl.ds(0, n)], sh_agg.at[pl.ds(row0, n)])
            row0 += n
        plsc.subcore_barrier()

        table = u_hbm.at[c] if channel_split else u_hbm
        wid = s if channel_split else s * NC + c

        def body(k, _):
            base = wid * ept + k * SCH
            pltpu.sync_copy(src_hbm.at[pl.ds(base, SCH)], idx_s)
            pltpu.sync_copy(dst_hbm.at[pl.ds(base, SCH)], idx_d)
            pltpu.sync_copy(table.at[idx_s], rows)
            pltpu.sync_copy(rows, sh_agg.at[idx_d], add=True)
            return 0

        lax.fori_loop(0, nchunk, body, 0)
        plsc.subcore_barrier()

        row0 = s * NPT
        for n in ROW_CHUNKS:
            pltpu.sync_copy(sh_agg.at[pl.ds(row0, n)], stage.at[pl.ds(0, n)])
            pltpu.sync_copy(stage.at[pl.ds(0, n)],
                            agg_hbm.at[c].at[pl.ds(row0, n)])
            row0 += n

    return spmm


_spmm_l1 = _make_spmm(True)
_spmm_l2 = _make_spmm(False)


BLK = 1264


def _tc1(x, W1, degp):
    def body(x_ref, w_ref, deg_ref, u_ref, dinv_ref):
        deg = deg_ref[0, :, 0:1] + deg_ref[1, :, 0:1] + 1.0
        dinv = lax.rsqrt(deg)
        h = jnp.dot(x_ref[...], w_ref[...], preferred_element_type=jnp.float32)
        u = h * dinv
        u_ref[0] = u[:, :128]
        u_ref[1] = u[:, 128:]
        dinv_ref[...] = dinv

    return pl.pallas_call(
        body,
        grid=(NPAD // BLK,),
        in_specs=[
            pl.BlockSpec((BLK, 128), lambda i: (i, 0)),
            pl.BlockSpec((128, 256), lambda i: (0, 0)),
            pl.BlockSpec((2, BLK, 128), lambda i: (0, i, 0)),
        ],
        out_specs=[
            pl.BlockSpec((2, BLK, 128), lambda i: (0, i, 0)),
            pl.BlockSpec((BLK, 1), lambda i: (i, 0)),
        ],
        out_shape=[
            jax.ShapeDtypeStruct((2, NPAD, 128), jnp.float32),
            jax.ShapeDtypeStruct((NPAD, 1), jnp.float32),
        ],
    )(x, W1, degp)


def _tc2(agg1, u1, dinv, b1, W2):
    def body(agg_ref, u_ref, dinv_ref, b_ref, w_ref, u2_ref):
        dinv = dinv_ref[...]
        m = jnp.concatenate(
            [agg_ref[0] + u_ref[0], agg_ref[1] + u_ref[1]], axis=1)
        z = jnp.maximum(dinv * m + b_ref[...], 0.0)
        h2 = jnp.dot(z, w_ref[...], preferred_element_type=jnp.float32)
        u2_ref[...] = dinv * h2

    return pl.pallas_call(
        body,
        grid=(NPAD // BLK,),
        in_specs=[
            pl.BlockSpec((2, BLK, 128), lambda i: (0, i, 0)),
            pl.BlockSpec((2, BLK, 128), lambda i: (0, i, 0)),
            pl.BlockSpec((BLK, 1), lambda i: (i, 0)),
            pl.BlockSpec((1, 256), lambda i: (0, 0)),
            pl.BlockSpec((256, 128), lambda i: (0, 0)),
        ],
        out_specs=pl.BlockSpec((BLK, 128), lambda i: (i, 0)),
        out_shape=jax.ShapeDtypeStruct((NPAD, 128), jnp.float32),
    )(agg1, u1, dinv, b1, W2)


def _tc3(agg2, u2, dinv, b2):
    def body(agg_ref, u_ref, dinv_ref, b_ref, out_ref):
        m = agg_ref[0] + agg_ref[1] + u_ref[...]
        out_ref[...] = dinv_ref[...] * m + b_ref[...]

    return pl.pallas_call(
        body,
        grid=(NPAD // BLK,),
        in_specs=[
            pl.BlockSpec((2, BLK, 128), lambda i: (0, i, 0)),
            pl.BlockSpec((BLK, 128), lambda i: (i, 0)),
            pl.BlockSpec((BLK, 1), lambda i: (i, 0)),
            pl.BlockSpec((1, 128), lambda i: (0, 0)),
        ],
        out_specs=pl.BlockSpec((BLK, 128), lambda i: (i, 0)),
        out_shape=jax.ShapeDtypeStruct((NPAD, 128), jnp.float32),
    )(agg2, u2, dinv, b2)


def kernel(x, edge_index, W1, b1, W2, b2):
    src = edge_index[0].astype(jnp.int32)
    dst = edge_index[1].astype(jnp.int32)
    xp = jnp.concatenate(
        [x, jnp.zeros((NPAD - N, x.shape[1]), x.dtype)], axis=0)
    degp = _deg_kernel(dst)
    u1, dinv = _tc1(xp, W1, degp)
    agg1 = _spmm_l1(u1, src, dst)
    u2 = _tc2(agg1, u1, dinv, b1.reshape(1, -1), W2)
    agg2 = _spmm_l2(u2, src, dst)
    return _tc3(agg2, u2, dinv, b2.reshape(1, -1))[:N]
